# baseline (speedup 1.0000x reference)
import jax
import jax.numpy as jnp
from jax import lax
from jax.experimental import pallas as pl
from jax.experimental.pallas import tpu as pltpu
from jax.experimental.pallas import tpu_sc as plsc

K = 64
N_ROWS = 128
N_COLS = 32768
NC = 2
NS = 16
NW = NC * NS
ROWS_PER_W = N_ROWS // NW
VREGS = N_COLS // 16
GROUPS = VREGS // 4
SURV = N_COLS + 128


def _sortd(v):
    s, _ = plsc.sort_key_val(v, v, descending=True)
    return s


def _rev(v):
    return lax.rev(v, (0,))


def _merge2(a, b):
    br = _rev(b)
    hi = jnp.maximum(a, br)
    lo = jnp.minimum(a, br)
    return _sortd(hi), _sortd(lo)


def _merge32(a0, a1, b0, b1):
    rb0, rb1 = _rev(b1), _rev(b0)
    hi0 = jnp.maximum(a0, rb0)
    hi1 = jnp.maximum(a1, rb1)
    lo0 = jnp.minimum(a0, rb0)
    lo1 = jnp.minimum(a1, rb1)
    h0 = jnp.maximum(hi0, hi1)
    h1 = jnp.minimum(hi0, hi1)
    l0 = jnp.maximum(lo0, lo1)
    l1 = jnp.minimum(lo0, lo1)
    return _sortd(h0), _sortd(h1), _sortd(l0), _sortd(l1)


def _sort64(c0, c1, c2, c3):
    a0, a1 = _merge2(_sortd(c0), _sortd(c1))
    b0, b1 = _merge2(_sortd(c2), _sortd(c3))
    return _merge32(a0, a1, b0, b1)


def _merge_top64(t, c):
    t0, t1, t2, t3 = t
    c0, c1, c2, c3 = c
    h0 = jnp.maximum(t0, _rev(c3))
    h1 = jnp.maximum(t1, _rev(c2))
    h2 = jnp.maximum(t2, _rev(c1))
    h3 = jnp.maximum(t3, _rev(c0))
    p0 = jnp.maximum(h0, h2)
    p2 = jnp.minimum(h0, h2)
    p1 = jnp.maximum(h1, h3)
    p3 = jnp.minimum(h1, h3)
    q0 = jnp.maximum(p0, p1)
    q1 = jnp.minimum(p0, p1)
    q2 = jnp.maximum(p2, p3)
    q3 = jnp.minimum(p2, p3)
    return _sortd(q0), _sortd(q1), _sortd(q2), _sortd(q3)


def _row_top64(row_v, surv_v):
    neg = jnp.full((16,), -jnp.inf, jnp.float32)

    def p1_body(g, r):
        r0, r1, r2, r3 = r
        base = g * 64
        m = jnp.maximum(
            jnp.maximum(row_v[pl.ds(base, 16)], row_v[pl.ds(base + 16, 16)]),
            jnp.maximum(row_v[pl.ds(base + 32, 16)],
                        row_v[pl.ds(base + 48, 16)]))
        n0 = jnp.maximum(r0, m)
        x = jnp.minimum(r0, m)
        n1 = jnp.maximum(r1, x)
        x = jnp.minimum(r1, x)
        n2 = jnp.maximum(r2, x)
        x = jnp.minimum(r2, x)
        n3 = jnp.maximum(r3, x)
        return n0, n1, n2, n3

    r = lax.fori_loop(0, GROUPS, p1_body, (neg, neg, neg, neg))
    tau = jnp.full((16,), jnp.min(r[3]), jnp.float32)

    def p2_body(j, off):
        base = j * 64
        for t in range(4):
            v = row_v[pl.ds(base + 16 * t, 16)]
            mask = v >= tau
            cnt = jnp.sum(jnp.where(mask, 1, 0))
            plsc.store_compressed(surv_v.at[pl.ds(off, 16)], v, mask=mask)
            off = off + cnt
        return off

    m_cnt = lax.fori_loop(0, GROUPS, p2_body, jnp.int32(0))

    for t in range(5):
        surv_v[pl.ds(m_cnt + 16 * t, 16)] = neg

    nblk = (m_cnt + 63) // 64

    def p3_body(c, t):
        base = c * 64
        c0 = surv_v[pl.ds(base, 16)]
        c1 = surv_v[pl.ds(base + 16, 16)]
        c2 = surv_v[pl.ds(base + 32, 16)]
        c3 = surv_v[pl.ds(base + 48, 16)]
        return _merge_top64(t, _sort64(c0, c1, c2, c3))

    return lax.fori_loop(0, nblk, p3_body, (neg, neg, neg, neg))


def _tec_body(x_hbm, out_hbm, row_v, surv_v, out_v):
    wid = lax.axis_index("s") * NC + lax.axis_index("c")
    row0 = wid * ROWS_PER_W
    for i in range(ROWS_PER_W):
        pltpu.sync_copy(x_hbm.at[row0 + i], row_v)
        t = _row_top64(row_v, surv_v)
        for k in range(4):
            out_v[i, pl.ds(16 * k, 16)] = t[k]
    pltpu.sync_copy(out_v, out_hbm.at[pl.ds(row0, ROWS_PER_W)])


def kernel(x):
    mesh = plsc.VectorSubcoreMesh(core_axis_name="c", subcore_axis_name="s")
    run = pl.kernel(
        _tec_body,
        mesh=mesh,
        out_type=jax.ShapeDtypeStruct((N_ROWS, K), jnp.float32),
        scratch_types=[
            pltpu.VMEM((N_COLS,), jnp.float32),
            pltpu.VMEM((SURV,), jnp.float32),
            pltpu.VMEM((ROWS_PER_W, K), jnp.float32),
        ],
        compiler_params=pltpu.CompilerParams(needs_layout_passes=False),
    )
    return run(x)


# speedup vs baseline: 2.9873x; 2.9873x over previous
import jax
import jax.numpy as jnp
from jax import lax
from jax.experimental import pallas as pl
from jax.experimental.pallas import tpu as pltpu
from jax.experimental.pallas import tpu_sc as plsc

K = 64
N_ROWS = 128
N_COLS = 32768
NC = 2
NS = 16
NW = NC * NS
ROWS_PER_W = N_ROWS // NW
VREGS = N_COLS // 16
GROUPS = VREGS // 4
SURV = N_COLS


def _sortd(v):
    s, _ = plsc.sort_key_val(v, v, descending=True)
    return s


def _rev(v):
    return lax.rev(v, (0,))


def _merge2(a, b):
    br = _rev(b)
    hi = jnp.maximum(a, br)
    lo = jnp.minimum(a, br)
    return _sortd(hi), _sortd(lo)


def _merge32(a0, a1, b0, b1):
    rb0, rb1 = _rev(b1), _rev(b0)
    hi0 = jnp.maximum(a0, rb0)
    hi1 = jnp.maximum(a1, rb1)
    lo0 = jnp.minimum(a0, rb0)
    lo1 = jnp.minimum(a1, rb1)
    h0 = jnp.maximum(hi0, hi1)
    h1 = jnp.minimum(hi0, hi1)
    l0 = jnp.maximum(lo0, lo1)
    l1 = jnp.minimum(lo0, lo1)
    return _sortd(h0), _sortd(h1), _sortd(l0), _sortd(l1)


def _sort64(c0, c1, c2, c3):
    a0, a1 = _merge2(_sortd(c0), _sortd(c1))
    b0, b1 = _merge2(_sortd(c2), _sortd(c3))
    return _merge32(a0, a1, b0, b1)


def _merge_top64(t, c):
    t0, t1, t2, t3 = t
    c0, c1, c2, c3 = c
    h0 = jnp.maximum(t0, _rev(c3))
    h1 = jnp.maximum(t1, _rev(c2))
    h2 = jnp.maximum(t2, _rev(c1))
    h3 = jnp.maximum(t3, _rev(c0))
    p0 = jnp.maximum(h0, h2)
    p2 = jnp.minimum(h0, h2)
    p1 = jnp.maximum(h1, h3)
    p3 = jnp.minimum(h1, h3)
    q0 = jnp.maximum(p0, p1)
    q1 = jnp.minimum(p0, p1)
    q2 = jnp.maximum(p2, p3)
    q3 = jnp.minimum(p2, p3)
    return _sortd(q0), _sortd(q1), _sortd(q2), _sortd(q3)


def _row_top64(row_v, surv_v):
    neg = jnp.full((16,), -jnp.inf, jnp.float32)

    def p1_body(gg, r):
        r0, r1, r2, r3 = r
        base0 = gg * 256
        for g in range(4):
            base = base0 + g * 64
            m = jnp.maximum(
                jnp.maximum(row_v[pl.ds(base, 16)],
                            row_v[pl.ds(base + 16, 16)]),
                jnp.maximum(row_v[pl.ds(base + 32, 16)],
                            row_v[pl.ds(base + 48, 16)]))
            n0 = jnp.maximum(r0, m)
            x = jnp.minimum(r0, m)
            n1 = jnp.maximum(r1, x)
            x = jnp.minimum(r1, x)
            n2 = jnp.maximum(r2, x)
            x = jnp.minimum(r2, x)
            n3 = jnp.maximum(r3, x)
            r0, r1, r2, r3 = n0, n1, n2, n3
        return r0, r1, r2, r3

    r = lax.fori_loop(0, GROUPS // 4, p1_body, (neg, neg, neg, neg))
    tau = jnp.full((16,), jnp.min(r[3]), jnp.float32)

    iota = jax.lax.iota(jnp.int32, 16)

    def p2_body(j, cnt16):
        base = j * 256
        vs, masks, idxs = [], [], []
        for t in range(16):
            v = row_v[pl.ds(base + 16 * t, 16)]
            mask = v >= tau
            vs.append(v)
            masks.append(mask)
            idxs.append(cnt16 + iota)
            cnt16 = cnt16 + jnp.where(mask, 16, 0)
        for t in range(16):
            plsc.store_scatter(surv_v, [idxs[t]], vs[t], mask=masks[t])
        return cnt16

    cnt16 = lax.fori_loop(0, GROUPS // 4, p2_body,
                          jnp.zeros((16,), jnp.int32))

    maxc = jnp.max(cnt16)
    nblk = (maxc + 48) // 64

    def p3_body(c, t):
        vs = []
        for u in range(4):
            d = c * 4 + u
            v = surv_v[pl.ds(d * 16, 16)]
            vs.append(jnp.where(d * 16 < cnt16, v, neg))
        return _merge_top64(t, _sort64(vs[0], vs[1], vs[2], vs[3]))

    return lax.fori_loop(0, nblk, p3_body, (neg, neg, neg, neg))


def _tec_body(x_hbm, out_hbm, row_a, row_b, surv_v, out_v, sem_a, sem_b):
    wid = lax.axis_index("s") * NC + lax.axis_index("c")
    row0 = wid * ROWS_PER_W
    bufs = (row_a, row_b)
    sems = (sem_a, sem_b)
    copies = [pltpu.async_copy(x_hbm.at[row0], row_a, sem_a)]
    for i in range(ROWS_PER_W):
        if i + 1 < ROWS_PER_W:
            copies.append(pltpu.async_copy(
                x_hbm.at[row0 + i + 1], bufs[(i + 1) % 2], sems[(i + 1) % 2]))
        copies[i].wait()
        t = _row_top64(bufs[i % 2], surv_v)
        for k in range(4):
            out_v[i, pl.ds(16 * k, 16)] = t[k]
    pltpu.sync_copy(out_v, out_hbm.at[pl.ds(row0, ROWS_PER_W)])


def kernel(x):
    mesh = plsc.VectorSubcoreMesh(core_axis_name="c", subcore_axis_name="s")
    run = pl.kernel(
        _tec_body,
        mesh=mesh,
        out_type=jax.ShapeDtypeStruct((N_ROWS, K), jnp.float32),
        scratch_types=[
            pltpu.VMEM((N_COLS,), jnp.float32),
            pltpu.VMEM((N_COLS,), jnp.float32),
            pltpu.VMEM((SURV,), jnp.float32),
            pltpu.VMEM((ROWS_PER_W, K), jnp.float32),
            pltpu.SemaphoreType.DMA,
            pltpu.SemaphoreType.DMA,
        ],
        compiler_params=pltpu.CompilerParams(needs_layout_passes=False),
    )
    return run(x)
